# baseline (speedup 1.0000x reference)
import functools

import jax
import jax.numpy as jnp
from jax.experimental import pallas as pl
from jax.experimental.pallas import tpu as pltpu

N = 2048
D_IN = 512
B_HID = 4096
CODE = 64
NUM_GRIDS = 8
GRID_MIN, GRID_MAX = -2.0, 2.0
BM = 128
NT = N // BM

_BF = jnp.bfloat16


def _dot(a, b):
    return jnp.dot(a, b, preferred_element_type=jnp.float32)


def _body(x_hbm, G_hbm, W1_hbm, W2_hbm, W3p_hbm,
          b1_ref, b2_ref, lnw_ref, lnb_ref, b3_ref,
          code_hbm, out_hbm,
          xv, W1v, W2v, W3v, gt0, gt1, xb, W1b, W2b, W3b, Gb, u,
          codet0, codet1, outt0, outt1,
          sx, sw1, sw2, sw3, sg0, sg1, so0, so1):
    gts = (gt0, gt1)
    sgs = (sg0, sg1)
    codets = (codet0, codet1)
    outts = (outt0, outt1)
    sos = (so0, so1)

    cp_x = pltpu.make_async_copy(x_hbm, xv, sx)
    cp_w1 = pltpu.make_async_copy(W1_hbm, W1v, sw1)
    cp_w2 = pltpu.make_async_copy(W2_hbm, W2v, sw2)
    cp_w3 = pltpu.make_async_copy(W3p_hbm, W3v, sw3)
    cp_g = [pltpu.make_async_copy(G_hbm.at[pl.ds(i * BM, BM), :], gts[i % 2], sgs[i % 2])
            for i in range(NT)]
    cp_g[0].start()
    cp_x.start()
    cp_g[1].start()
    cp_w1.start()
    cp_w2.start()
    cp_w3.start()

    cp_x.wait()
    xb[...] = xv[...].astype(_BF)

    for i in range(NT):
        cp_g[i].wait()
        gbt = gts[i % 2][...].astype(_BF)
        Gb[pl.ds(i * BM, BM), :] = gbt
        if i + 2 < NT:
            cp_g[i + 2].start()
        t = _dot(gbt, xb[...])
        if i == 0:
            cp_w1.wait()
            W1b[...] = W1v[...].astype(_BF)
        h = jnp.maximum(_dot(t.astype(_BF), W1b[...]) + b1_ref[...], 0.0)
        if i == 0:
            cp_w2.wait()
            W2b[...] = W2v[...].astype(_BF)
        u[pl.ds(i * BM, BM), :] = _dot(h.astype(_BF), W2b[...]).astype(_BF)

    cp_w3.wait()
    W3b[...] = W3v[...].astype(_BF)

    denom = (GRID_MAX - GRID_MIN) / (NUM_GRIDS - 1)
    out_cps = []
    for i in range(NT):
        if i >= 2:
            out_cps[i - 2][0].wait()
            out_cps[i - 2][1].wait()
        gt = Gb[pl.ds(i * BM, BM), :]
        feat = _dot(gt, u[...]) + b2_ref[...]
        code = jnp.tanh(10.0 * feat)
        codets[i % 2][...] = code
        mu = jnp.mean(code, axis=-1, keepdims=True)
        var = jnp.mean((code - mu) ** 2, axis=-1, keepdims=True)
        y = (code - mu) * jax.lax.rsqrt(var + 1e-5) * lnw_ref[...] + lnb_ref[...]
        rbf_blocks = []
        for g in range(NUM_GRIDS):
            gval = GRID_MIN + denom * g
            rbf_blocks.append(jnp.exp(-(((y - gval) / denom) ** 2)))
        rbf = jnp.concatenate(rbf_blocks, axis=-1)
        out = _dot(rbf.astype(_BF), W3b[...]) + b3_ref[...]
        outts[i % 2][...] = jnp.maximum(out, 0.0)
        cpc = pltpu.make_async_copy(codets[i % 2], code_hbm.at[pl.ds(i * BM, BM), :], sos[i % 2])
        cpo = pltpu.make_async_copy(outts[i % 2], out_hbm.at[pl.ds(i * BM, BM), :], sos[i % 2])
        cpc.start()
        cpo.start()
        out_cps.append((cpc, cpo))

    for i in range(NT - 2, NT):
        out_cps[i][0].wait()
        out_cps[i][1].wait()


def kernel(x, G, W1, b1, W2, b2, ln_w, ln_b, W3, b3):
    b1r = b1.reshape(1, B_HID)
    b2r = b2.reshape(1, CODE)
    lnwr = ln_w.reshape(1, CODE)
    lnbr = ln_b.reshape(1, CODE)
    b3r = b3.reshape(1, 2 * D_IN)
    W3p = W3.reshape(CODE, NUM_GRIDS, 2 * D_IN).transpose(1, 0, 2) \
             .reshape(NUM_GRIDS * CODE, 2 * D_IN)

    any_spec = pl.BlockSpec(memory_space=pl.ANY)

    def vsmall(shape):
        return pl.BlockSpec(shape, lambda: tuple(0 for _ in shape))

    code, feat_out = pl.pallas_call(
        _body,
        in_specs=[
            any_spec, any_spec, any_spec, any_spec, any_spec,
            vsmall((1, B_HID)), vsmall((1, CODE)), vsmall((1, CODE)),
            vsmall((1, CODE)), vsmall((1, 2 * D_IN)),
        ],
        out_specs=[any_spec, any_spec],
        out_shape=[
            jax.ShapeDtypeStruct((N, CODE), jnp.float32),
            jax.ShapeDtypeStruct((N, 2 * D_IN), jnp.float32),
        ],
        scratch_shapes=[
            pltpu.VMEM((N, D_IN), jnp.float32),
            pltpu.VMEM((D_IN, B_HID), jnp.float32),
            pltpu.VMEM((B_HID, CODE), jnp.float32),
            pltpu.VMEM((NUM_GRIDS * CODE, 2 * D_IN), jnp.float32),
            pltpu.VMEM((BM, N), jnp.float32),
            pltpu.VMEM((BM, N), jnp.float32),
            pltpu.VMEM((N, D_IN), _BF),
            pltpu.VMEM((D_IN, B_HID), _BF),
            pltpu.VMEM((B_HID, CODE), _BF),
            pltpu.VMEM((NUM_GRIDS * CODE, 2 * D_IN), _BF),
            pltpu.VMEM((N, N), _BF),
            pltpu.VMEM((N, CODE), _BF),
            pltpu.VMEM((BM, CODE), jnp.float32),
            pltpu.VMEM((BM, CODE), jnp.float32),
            pltpu.VMEM((BM, 2 * D_IN), jnp.float32),
            pltpu.VMEM((BM, 2 * D_IN), jnp.float32),
            pltpu.SemaphoreType.DMA,
            pltpu.SemaphoreType.DMA,
            pltpu.SemaphoreType.DMA,
            pltpu.SemaphoreType.DMA,
            pltpu.SemaphoreType.DMA,
            pltpu.SemaphoreType.DMA,
            pltpu.SemaphoreType.DMA,
            pltpu.SemaphoreType.DMA,
        ],
    )(x, G, W1, W2, W3p, b1r, b2r, lnwr, lnbr, b3r)

    return (code, feat_out)


# speedup vs baseline: 1.0444x; 1.0444x over previous
import functools

import jax
import jax.numpy as jnp
from jax.experimental import pallas as pl
from jax.experimental.pallas import tpu as pltpu

N = 2048
D_IN = 512
B_HID = 4096
CODE = 64
NUM_GRIDS = 8
GRID_MIN, GRID_MAX = -2.0, 2.0
BM = 256
NT = N // BM

_BF = jnp.bfloat16


def _dot(a, b):
    return jnp.dot(a, b, preferred_element_type=jnp.float32)


def _body(x_hbm, G_hbm, W1_hbm, W2_hbm, W3p_hbm,
          b1_ref, b2_ref, lnw_ref, lnb_ref, b3_ref,
          code_hbm, out_hbm,
          xv, W1v, W2v, W3v, gt0, gt1, xb, W1b, W2b, W3b, Gb, u,
          codet0, codet1, outt0, outt1,
          sx, sw1, sw2, sw3, sg0, sg1, so0, so1):
    gts = (gt0, gt1)
    sgs = (sg0, sg1)
    codets = (codet0, codet1)
    outts = (outt0, outt1)
    sos = (so0, so1)

    cp_x = pltpu.make_async_copy(x_hbm, xv, sx)
    cp_w1 = pltpu.make_async_copy(W1_hbm, W1v, sw1)
    cp_w2 = pltpu.make_async_copy(W2_hbm, W2v, sw2)
    cp_w3 = pltpu.make_async_copy(W3p_hbm, W3v, sw3)
    cp_g = [pltpu.make_async_copy(G_hbm.at[pl.ds(i * BM, BM), :], gts[i % 2], sgs[i % 2])
            for i in range(NT)]
    cp_g[0].start()
    cp_x.start()
    cp_g[1].start()
    cp_w1.start()
    cp_w2.start()
    cp_w3.start()

    cp_x.wait()
    xb[...] = xv[...].astype(_BF)

    for i in range(NT):
        cp_g[i].wait()
        gbt = gts[i % 2][...].astype(_BF)
        Gb[pl.ds(i * BM, BM), :] = gbt
        if i + 2 < NT:
            cp_g[i + 2].start()
        t = _dot(gbt, xb[...])
        if i == 0:
            cp_w1.wait()
            W1b[...] = W1v[...].astype(_BF)
            cp_w2.wait()
            W2b[...] = W2v[...].astype(_BF)
        tb = t.astype(_BF)
        uacc = None
        for j in range(8):
            sl = pl.ds(j * (B_HID // 8), B_HID // 8)
            hj = jnp.maximum(_dot(tb, W1b[:, sl]) + b1_ref[:, sl], 0.0)
            pj = _dot(hj.astype(_BF), W2b[sl, :])
            uacc = pj if uacc is None else uacc + pj
        u[pl.ds(i * BM, BM), :] = uacc.astype(_BF)

    cp_w3.wait()
    W3b[...] = W3v[...].astype(_BF)

    denom = (GRID_MAX - GRID_MIN) / (NUM_GRIDS - 1)
    out_cps = []
    for i in range(NT):
        if i >= 2:
            out_cps[i - 2][0].wait()
            out_cps[i - 2][1].wait()
        gt = Gb[pl.ds(i * BM, BM), :]
        feat = _dot(gt, u[...]) + b2_ref[...]
        code = jnp.tanh(10.0 * feat)
        codets[i % 2][...] = code
        mu = jnp.mean(code, axis=-1, keepdims=True)
        var = jnp.mean((code - mu) ** 2, axis=-1, keepdims=True)
        y = (code - mu) * jax.lax.rsqrt(var + 1e-5) * lnw_ref[...] + lnb_ref[...]
        rbf_blocks = []
        for g in range(NUM_GRIDS):
            gval = GRID_MIN + denom * g
            rbf_blocks.append(jnp.exp(-(((y - gval) / denom) ** 2)))
        rbf = jnp.concatenate(rbf_blocks, axis=-1)
        out = _dot(rbf.astype(_BF), W3b[...]) + b3_ref[...]
        outts[i % 2][...] = jnp.maximum(out, 0.0)
        cpc = pltpu.make_async_copy(codets[i % 2], code_hbm.at[pl.ds(i * BM, BM), :], sos[i % 2])
        cpo = pltpu.make_async_copy(outts[i % 2], out_hbm.at[pl.ds(i * BM, BM), :], sos[i % 2])
        cpc.start()
        cpo.start()
        out_cps.append((cpc, cpo))

    for i in range(NT - 2, NT):
        out_cps[i][0].wait()
        out_cps[i][1].wait()


def kernel(x, G, W1, b1, W2, b2, ln_w, ln_b, W3, b3):
    b1r = b1.reshape(1, B_HID)
    b2r = b2.reshape(1, CODE)
    lnwr = ln_w.reshape(1, CODE)
    lnbr = ln_b.reshape(1, CODE)
    b3r = b3.reshape(1, 2 * D_IN)
    W3p = W3.reshape(CODE, NUM_GRIDS, 2 * D_IN).transpose(1, 0, 2) \
             .reshape(NUM_GRIDS * CODE, 2 * D_IN)

    any_spec = pl.BlockSpec(memory_space=pl.ANY)

    def vsmall(shape):
        return pl.BlockSpec(shape, lambda: tuple(0 for _ in shape))

    code, feat_out = pl.pallas_call(
        _body,
        in_specs=[
            any_spec, any_spec, any_spec, any_spec, any_spec,
            vsmall((1, B_HID)), vsmall((1, CODE)), vsmall((1, CODE)),
            vsmall((1, CODE)), vsmall((1, 2 * D_IN)),
        ],
        out_specs=[any_spec, any_spec],
        out_shape=[
            jax.ShapeDtypeStruct((N, CODE), jnp.float32),
            jax.ShapeDtypeStruct((N, 2 * D_IN), jnp.float32),
        ],
        scratch_shapes=[
            pltpu.VMEM((N, D_IN), jnp.float32),
            pltpu.VMEM((D_IN, B_HID), jnp.float32),
            pltpu.VMEM((B_HID, CODE), jnp.float32),
            pltpu.VMEM((NUM_GRIDS * CODE, 2 * D_IN), jnp.float32),
            pltpu.VMEM((BM, N), jnp.float32),
            pltpu.VMEM((BM, N), jnp.float32),
            pltpu.VMEM((N, D_IN), _BF),
            pltpu.VMEM((D_IN, B_HID), _BF),
            pltpu.VMEM((B_HID, CODE), _BF),
            pltpu.VMEM((NUM_GRIDS * CODE, 2 * D_IN), _BF),
            pltpu.VMEM((N, N), _BF),
            pltpu.VMEM((N, CODE), _BF),
            pltpu.VMEM((BM, CODE), jnp.float32),
            pltpu.VMEM((BM, CODE), jnp.float32),
            pltpu.VMEM((BM, 2 * D_IN), jnp.float32),
            pltpu.VMEM((BM, 2 * D_IN), jnp.float32),
            pltpu.SemaphoreType.DMA,
            pltpu.SemaphoreType.DMA,
            pltpu.SemaphoreType.DMA,
            pltpu.SemaphoreType.DMA,
            pltpu.SemaphoreType.DMA,
            pltpu.SemaphoreType.DMA,
            pltpu.SemaphoreType.DMA,
            pltpu.SemaphoreType.DMA,
        ],
    )(x, G, W1, W2, W3p, b1r, b2r, lnwr, lnbr, b3r)

    return (code, feat_out)


# speedup vs baseline: 1.1596x; 1.1103x over previous
import functools

import jax
import jax.numpy as jnp
from jax.experimental import pallas as pl
from jax.experimental.pallas import tpu as pltpu

N = 2048
D_IN = 512
B_HID = 4096
CODE = 64
NUM_GRIDS = 8
GRID_MIN, GRID_MAX = -2.0, 2.0
BM = 256
NT = N // BM

_BF = jnp.bfloat16


def _dot(a, b):
    return jnp.dot(a, b, preferred_element_type=jnp.float32)


def _body(x_hbm, G_hbm, W1_hbm, W2_hbm, W3p_hbm,
          b1_ref, b2_ref, lnw_ref, lnb_ref, b3_ref,
          code_hbm, out_hbm,
          xv, W1v, W2v, W3v, gt0, gt1, gt2, gt3, xb, W1b, W2b, W3b, Gb, u,
          codet0, codet1, outt0, outt1,
          sx, sw1, sw2, sw3, sg0, sg1, sg2, sg3, so0, so1):
    gts = (gt0, gt1, gt2, gt3)
    sgs = (sg0, sg1, sg2, sg3)
    codets = (codet0, codet1)
    outts = (outt0, outt1)
    sos = (so0, so1)

    cp_x = pltpu.make_async_copy(x_hbm, xv, sx)
    cp_w1 = pltpu.make_async_copy(W1_hbm, W1v, sw1)
    cp_w2 = pltpu.make_async_copy(W2_hbm, W2v, sw2)
    cp_w3 = pltpu.make_async_copy(W3p_hbm, W3v, sw3)
    cp_g = [pltpu.make_async_copy(G_hbm.at[pl.ds(i * BM, BM), :], gts[i % 4], sgs[i % 4])
            for i in range(NT)]
    cp_g[0].start()
    cp_x.start()
    cp_w1.start()
    cp_g[1].start()
    cp_g[2].start()
    cp_g[3].start()
    cp_w2.start()
    cp_w3.start()

    cp_x.wait()
    xb[...] = xv[...].astype(_BF)

    for i in range(NT):
        cp_g[i].wait()
        gbt = gts[i % 4][...].astype(_BF)
        Gb[pl.ds(i * BM, BM), :] = gbt
        if i + 4 < NT:
            cp_g[i + 4].start()
        t = _dot(gbt, xb[...])
        if i == 0:
            cp_w1.wait()
            W1b[...] = W1v[...].astype(_BF)
        h = jnp.maximum(_dot(t.astype(_BF), W1b[...]) + b1_ref[...], 0.0)
        if i == 0:
            cp_w2.wait()
            W2b[...] = W2v[...].astype(_BF)
        u[pl.ds(i * BM, BM), :] = _dot(h.astype(_BF), W2b[...]).astype(_BF)

    cp_w3.wait()
    W3b[...] = W3v[...].astype(_BF)

    denom = (GRID_MAX - GRID_MIN) / (NUM_GRIDS - 1)
    out_cps = []
    for i in range(NT):
        if i >= 2:
            out_cps[i - 2][0].wait()
            out_cps[i - 2][1].wait()
        gt = Gb[pl.ds(i * BM, BM), :]
        feat = _dot(gt, u[...]) + b2_ref[...]
        code = jnp.tanh(10.0 * feat)
        codets[i % 2][...] = code
        mu = jnp.mean(code, axis=-1, keepdims=True)
        var = jnp.mean((code - mu) ** 2, axis=-1, keepdims=True)
        y = (code - mu) * jax.lax.rsqrt(var + 1e-5) * lnw_ref[...] + lnb_ref[...]
        rbf_blocks = []
        for g in range(NUM_GRIDS):
            gval = GRID_MIN + denom * g
            rbf_blocks.append(jnp.exp(-(((y - gval) / denom) ** 2)))
        rbf = jnp.concatenate(rbf_blocks, axis=-1)
        out = _dot(rbf.astype(_BF), W3b[...]) + b3_ref[...]
        outts[i % 2][...] = jnp.maximum(out, 0.0)
        cpc = pltpu.make_async_copy(codets[i % 2], code_hbm.at[pl.ds(i * BM, BM), :], sos[i % 2])
        cpo = pltpu.make_async_copy(outts[i % 2], out_hbm.at[pl.ds(i * BM, BM), :], sos[i % 2])
        cpc.start()
        cpo.start()
        out_cps.append((cpc, cpo))

    for i in range(NT - 2, NT):
        out_cps[i][0].wait()
        out_cps[i][1].wait()


def kernel(x, G, W1, b1, W2, b2, ln_w, ln_b, W3, b3):
    b1r = b1.reshape(1, B_HID)
    b2r = b2.reshape(1, CODE)
    lnwr = ln_w.reshape(1, CODE)
    lnbr = ln_b.reshape(1, CODE)
    b3r = b3.reshape(1, 2 * D_IN)
    W3p = W3.reshape(CODE, NUM_GRIDS, 2 * D_IN).transpose(1, 0, 2) \
             .reshape(NUM_GRIDS * CODE, 2 * D_IN)

    any_spec = pl.BlockSpec(memory_space=pl.ANY)

    def vsmall(shape):
        return pl.BlockSpec(shape, lambda: tuple(0 for _ in shape))

    code, feat_out = pl.pallas_call(
        _body,
        in_specs=[
            any_spec, any_spec, any_spec, any_spec, any_spec,
            vsmall((1, B_HID)), vsmall((1, CODE)), vsmall((1, CODE)),
            vsmall((1, CODE)), vsmall((1, 2 * D_IN)),
        ],
        out_specs=[any_spec, any_spec],
        out_shape=[
            jax.ShapeDtypeStruct((N, CODE), jnp.float32),
            jax.ShapeDtypeStruct((N, 2 * D_IN), jnp.float32),
        ],
        scratch_shapes=[
            pltpu.VMEM((N, D_IN), jnp.float32),
            pltpu.VMEM((D_IN, B_HID), jnp.float32),
            pltpu.VMEM((B_HID, CODE), jnp.float32),
            pltpu.VMEM((NUM_GRIDS * CODE, 2 * D_IN), jnp.float32),
            pltpu.VMEM((BM, N), jnp.float32),
            pltpu.VMEM((BM, N), jnp.float32),
            pltpu.VMEM((BM, N), jnp.float32),
            pltpu.VMEM((BM, N), jnp.float32),
            pltpu.VMEM((N, D_IN), _BF),
            pltpu.VMEM((D_IN, B_HID), _BF),
            pltpu.VMEM((B_HID, CODE), _BF),
            pltpu.VMEM((NUM_GRIDS * CODE, 2 * D_IN), _BF),
            pltpu.VMEM((N, N), _BF),
            pltpu.VMEM((N, CODE), _BF),
            pltpu.VMEM((BM, CODE), jnp.float32),
            pltpu.VMEM((BM, CODE), jnp.float32),
            pltpu.VMEM((BM, 2 * D_IN), jnp.float32),
            pltpu.VMEM((BM, 2 * D_IN), jnp.float32),
            pltpu.SemaphoreType.DMA,
            pltpu.SemaphoreType.DMA,
            pltpu.SemaphoreType.DMA,
            pltpu.SemaphoreType.DMA,
            pltpu.SemaphoreType.DMA,
            pltpu.SemaphoreType.DMA,
            pltpu.SemaphoreType.DMA,
            pltpu.SemaphoreType.DMA,
            pltpu.SemaphoreType.DMA,
            pltpu.SemaphoreType.DMA,
        ],
    )(x, G, W1, W2, W3p, b1r, b2r, lnwr, lnbr, b3r)

    return (code, feat_out)


# speedup vs baseline: 1.1765x; 1.0146x over previous
import functools

import jax
import jax.numpy as jnp
from jax.experimental import pallas as pl
from jax.experimental.pallas import tpu as pltpu

N = 2048
D_IN = 512
B_HID = 4096
CODE = 64
NUM_GRIDS = 8
GRID_MIN, GRID_MAX = -2.0, 2.0
BM = 256
NT = N // BM

_BF = jnp.bfloat16


def _dot(a, b):
    return jnp.dot(a, b, preferred_element_type=jnp.float32)


def _body(x_hbm, G_hbm, W1_hbm, W2_hbm, W3p_hbm,
          b1_ref, b2_ref, lnw_ref, lnb_ref, b3_ref,
          code_hbm, out_hbm,
          xv, W1v, W2v, W3v, gt0, gt1, xb, W1b, W2b, W3b, Gb, u,
          codet0, codet1, outt0, outt1,
          sx, sw1, sw2, sw3, sg0, sg1, so0, so1):
    gts = (gt0, gt1)
    sgs = (sg0, sg1)
    codets = (codet0, codet1)
    outts = (outt0, outt1)
    sos = (so0, so1)

    cp_x = pltpu.make_async_copy(x_hbm, xv, sx)
    cp_w1 = pltpu.make_async_copy(W1_hbm, W1v, sw1)
    cp_w2 = pltpu.make_async_copy(W2_hbm, W2v, sw2)
    cp_w3 = pltpu.make_async_copy(W3p_hbm, W3v, sw3)
    cp_g = [pltpu.make_async_copy(G_hbm.at[pl.ds(i * BM, BM), :], gts[i % 2], sgs[i % 2])
            for i in range(NT)]
    cp_g[0].start()
    cp_x.start()
    cp_g[1].start()
    cp_w1.start()
    cp_w2.start()
    cp_w3.start()

    cp_x.wait()
    xb[...] = xv[...].astype(_BF)

    for i in range(NT):
        cp_g[i].wait()
        gbt = gts[i % 2][...].astype(_BF)
        Gb[pl.ds(i * BM, BM), :] = gbt
        if i + 2 < NT:
            cp_g[i + 2].start()
        t = _dot(gbt, xb[...])
        if i == 0:
            cp_w1.wait()
            W1b[...] = W1v[...].astype(_BF)
        h = jnp.maximum(_dot(t.astype(_BF), W1b[...]) + b1_ref[...], 0.0)
        if i == 0:
            cp_w2.wait()
            W2b[...] = W2v[...].astype(_BF)
        u[pl.ds(i * BM, BM), :] = _dot(h.astype(_BF), W2b[...]).astype(_BF)

    cp_w3.wait()
    W3b[...] = W3v[...].astype(_BF)

    denom = (GRID_MAX - GRID_MIN) / (NUM_GRIDS - 1)
    out_cps = []
    for i in range(NT):
        if i >= 2:
            out_cps[i - 2][0].wait()
            out_cps[i - 2][1].wait()
        gt = Gb[pl.ds(i * BM, BM), :]
        feat = _dot(gt, u[...]) + b2_ref[...]
        code = jnp.tanh(10.0 * feat)
        codets[i % 2][...] = code
        mu = jnp.mean(code, axis=-1, keepdims=True)
        var = jnp.mean((code - mu) ** 2, axis=-1, keepdims=True)
        y = (code - mu) * jax.lax.rsqrt(var + 1e-5) * lnw_ref[...] + lnb_ref[...]
        rbf_blocks = []
        for g in range(NUM_GRIDS):
            gval = GRID_MIN + denom * g
            rbf_blocks.append(jnp.exp(-(((y - gval) / denom) ** 2)))
        rbf = jnp.concatenate(rbf_blocks, axis=-1)
        out = _dot(rbf.astype(_BF), W3b[...]) + b3_ref[...]
        outts[i % 2][...] = jnp.maximum(out, 0.0)
        cpc = pltpu.make_async_copy(codets[i % 2], code_hbm.at[pl.ds(i * BM, BM), :], sos[i % 2])
        cpo = pltpu.make_async_copy(outts[i % 2], out_hbm.at[pl.ds(i * BM, BM), :], sos[i % 2])
        cpc.start()
        cpo.start()
        out_cps.append((cpc, cpo))

    for i in range(NT - 2, NT):
        out_cps[i][0].wait()
        out_cps[i][1].wait()


def kernel(x, G, W1, b1, W2, b2, ln_w, ln_b, W3, b3):
    b1r = b1.reshape(1, B_HID)
    b2r = b2.reshape(1, CODE)
    lnwr = ln_w.reshape(1, CODE)
    lnbr = ln_b.reshape(1, CODE)
    b3r = b3.reshape(1, 2 * D_IN)
    W3p = W3.reshape(CODE, NUM_GRIDS, 2 * D_IN).transpose(1, 0, 2) \
             .reshape(NUM_GRIDS * CODE, 2 * D_IN)

    any_spec = pl.BlockSpec(memory_space=pl.ANY)

    def vsmall(shape):
        return pl.BlockSpec(shape, lambda: tuple(0 for _ in shape))

    code, feat_out = pl.pallas_call(
        _body,
        in_specs=[
            any_spec, any_spec, any_spec, any_spec, any_spec,
            vsmall((1, B_HID)), vsmall((1, CODE)), vsmall((1, CODE)),
            vsmall((1, CODE)), vsmall((1, 2 * D_IN)),
        ],
        out_specs=[any_spec, any_spec],
        out_shape=[
            jax.ShapeDtypeStruct((N, CODE), jnp.float32),
            jax.ShapeDtypeStruct((N, 2 * D_IN), jnp.float32),
        ],
        scratch_shapes=[
            pltpu.VMEM((N, D_IN), jnp.float32),
            pltpu.VMEM((D_IN, B_HID), jnp.float32),
            pltpu.VMEM((B_HID, CODE), jnp.float32),
            pltpu.VMEM((NUM_GRIDS * CODE, 2 * D_IN), jnp.float32),
            pltpu.VMEM((BM, N), jnp.float32),
            pltpu.VMEM((BM, N), jnp.float32),
            pltpu.VMEM((N, D_IN), _BF),
            pltpu.VMEM((D_IN, B_HID), _BF),
            pltpu.VMEM((B_HID, CODE), _BF),
            pltpu.VMEM((NUM_GRIDS * CODE, 2 * D_IN), _BF),
            pltpu.VMEM((N, N), _BF),
            pltpu.VMEM((N, CODE), _BF),
            pltpu.VMEM((BM, CODE), jnp.float32),
            pltpu.VMEM((BM, CODE), jnp.float32),
            pltpu.VMEM((BM, 2 * D_IN), jnp.float32),
            pltpu.VMEM((BM, 2 * D_IN), jnp.float32),
            pltpu.SemaphoreType.DMA,
            pltpu.SemaphoreType.DMA,
            pltpu.SemaphoreType.DMA,
            pltpu.SemaphoreType.DMA,
            pltpu.SemaphoreType.DMA,
            pltpu.SemaphoreType.DMA,
            pltpu.SemaphoreType.DMA,
            pltpu.SemaphoreType.DMA,
        ],
    )(x, G, W1, W2, W3p, b1r, b2r, lnwr, lnbr, b3r)

    return (code, feat_out)


# speedup vs baseline: 1.1806x; 1.0034x over previous
import jax
import jax.numpy as jnp
from jax.experimental import pallas as pl
from jax.experimental.pallas import tpu as pltpu

N = 2048
D_IN = 512
B_HID = 4096
CODE = 64
NUM_GRIDS = 8
GRID_MIN, GRID_MAX = -2.0, 2.0
BM = 256
NT = N // BM

_BF = jnp.bfloat16


def _dot(a, b):
    return jnp.dot(a, b, preferred_element_type=jnp.float32)


def _body(x_hbm, G_hbm, W1_hbm, W2_hbm, W3p_hbm,
          b1_ref, b2_ref, lnw_ref, lnb_ref, b3_ref,
          code_hbm, out_hbm,
          xv, W1v, W2v, W3v, gt0, gt1, xb, W1b, W2b, W3b, Gb, u,
          codet0, codet1, outt0, outt1,
          sx, sw1, sw2, sw3, sg0, sg1, so0, so1):
    gts = (gt0, gt1)
    sgs = (sg0, sg1)
    codets = (codet0, codet1)
    outts = (outt0, outt1)
    sos = (so0, so1)

    cp_x = pltpu.make_async_copy(x_hbm, xv, sx)
    cp_w1 = pltpu.make_async_copy(W1_hbm, W1v, sw1)
    cp_w2 = pltpu.make_async_copy(W2_hbm, W2v, sw2)
    cp_w3 = pltpu.make_async_copy(W3p_hbm, W3v, sw3)
    cp_g = [pltpu.make_async_copy(G_hbm.at[pl.ds(i * BM, BM), :], gts[i % 2], sgs[i % 2])
            for i in range(NT)]
    cp_g[0].start()
    cp_x.start()
    cp_g[1].start()
    cp_w1.start()
    cp_w2.start()
    cp_w3.start()

    cp_x.wait()
    xb[...] = xv[...].astype(_BF)

    for i in range(NT):
        cp_g[i].wait()
        gbt = gts[i % 2][...].astype(_BF)
        Gb[pl.ds(i * BM, BM), :] = gbt
        if i + 2 < NT:
            cp_g[i + 2].start()
        t = _dot(gbt, xb[...])
        if i == 0:
            cp_w1.wait()
            W1b[...] = W1v[...].astype(_BF)
        h = jnp.maximum(_dot(t.astype(_BF), W1b[...]) + b1_ref[...], 0.0)
        if i == 0:
            cp_w2.wait()
            W2b[...] = W2v[...].astype(_BF)
        u[pl.ds(i * BM, BM), :] = _dot(h.astype(_BF), W2b[...]).astype(_BF)

    cp_w3.wait()
    W3b[...] = W3v[...].astype(_BF)

    denom = (GRID_MAX - GRID_MIN) / (NUM_GRIDS - 1)
    out_cps = []
    for i in range(NT):
        if i >= 2:
            out_cps[i - 2][0].wait()
            out_cps[i - 2][1].wait()
        gt = Gb[pl.ds(i * BM, BM), :]
        feat = _dot(gt, u[...]) + b2_ref[...]
        code = jnp.tanh(10.0 * feat)
        codets[i % 2][...] = code
        mu = jnp.mean(code, axis=-1, keepdims=True)
        var = jnp.mean((code - mu) ** 2, axis=-1, keepdims=True)
        y = (code - mu) * jax.lax.rsqrt(var + 1e-5) * lnw_ref[...] + lnb_ref[...]
        rbf_blocks = []
        for g in range(NUM_GRIDS):
            gval = GRID_MIN + denom * g
            rbf_blocks.append(jnp.exp(-(((y - gval) / denom) ** 2)))
        rbf = jnp.concatenate(rbf_blocks, axis=-1)
        out = _dot(rbf.astype(_BF), W3b[...]) + b3_ref[...]
        outts[i % 2][...] = jnp.maximum(out, 0.0)
        cpc = pltpu.make_async_copy(codets[i % 2], code_hbm.at[pl.ds(i * BM, BM), :], sos[i % 2])
        cpo = pltpu.make_async_copy(outts[i % 2], out_hbm.at[pl.ds(i * BM, BM), :], sos[i % 2])
        cpc.start()
        cpo.start()
        out_cps.append((cpc, cpo))

    for i in range(NT - 2, NT):
        out_cps[i][0].wait()
        out_cps[i][1].wait()


def kernel(x, G, W1, b1, W2, b2, ln_w, ln_b, W3, b3):
    b1r = b1.reshape(1, B_HID)
    b2r = b2.reshape(1, CODE)
    lnwr = ln_w.reshape(1, CODE)
    lnbr = ln_b.reshape(1, CODE)
    b3r = b3.reshape(1, 2 * D_IN)
    W3p = W3.reshape(CODE, NUM_GRIDS, 2 * D_IN).transpose(1, 0, 2) \
             .reshape(NUM_GRIDS * CODE, 2 * D_IN)

    any_spec = pl.BlockSpec(memory_space=pl.ANY)

    def vsmall(shape):
        return pl.BlockSpec(shape, lambda: tuple(0 for _ in shape))

    code, feat_out = pl.pallas_call(
        _body,
        in_specs=[
            any_spec, any_spec, any_spec, any_spec, any_spec,
            vsmall((1, B_HID)), vsmall((1, CODE)), vsmall((1, CODE)),
            vsmall((1, CODE)), vsmall((1, 2 * D_IN)),
        ],
        out_specs=[any_spec, any_spec],
        out_shape=[
            jax.ShapeDtypeStruct((N, CODE), jnp.float32),
            jax.ShapeDtypeStruct((N, 2 * D_IN), jnp.float32),
        ],
        scratch_shapes=[
            pltpu.VMEM((N, D_IN), jnp.float32),
            pltpu.VMEM((D_IN, B_HID), jnp.float32),
            pltpu.VMEM((B_HID, CODE), jnp.float32),
            pltpu.VMEM((NUM_GRIDS * CODE, 2 * D_IN), jnp.float32),
            pltpu.VMEM((BM, N), jnp.float32),
            pltpu.VMEM((BM, N), jnp.float32),
            pltpu.VMEM((N, D_IN), _BF),
            pltpu.VMEM((D_IN, B_HID), _BF),
            pltpu.VMEM((B_HID, CODE), _BF),
            pltpu.VMEM((NUM_GRIDS * CODE, 2 * D_IN), _BF),
            pltpu.VMEM((N, N), _BF),
            pltpu.VMEM((N, CODE), _BF),
            pltpu.VMEM((BM, CODE), jnp.float32),
            pltpu.VMEM((BM, CODE), jnp.float32),
            pltpu.VMEM((BM, 2 * D_IN), jnp.float32),
            pltpu.VMEM((BM, 2 * D_IN), jnp.float32),
            pltpu.SemaphoreType.DMA,
            pltpu.SemaphoreType.DMA,
            pltpu.SemaphoreType.DMA,
            pltpu.SemaphoreType.DMA,
            pltpu.SemaphoreType.DMA,
            pltpu.SemaphoreType.DMA,
            pltpu.SemaphoreType.DMA,
            pltpu.SemaphoreType.DMA,
        ],
    )(x, G, W1, W2, W3p, b1r, b2r, lnwr, lnbr, b3r)

    return (code, feat_out)
